# baseline (speedup 1.0000x reference)
import functools

import jax
import jax.numpy as jnp
from jax import lax
from jax.experimental import pallas as pl
from jax.experimental.pallas import tpu as pltpu
from jax.experimental.pallas import tpu_sc as plsc

N_NODES = 10000
N_PAD = 10240
N_EDGES = 320000
E_PAD = 327680
NTILES = 16
NCORES = 2
ROWS_PER_TILE = N_PAD // NTILES
CHUNK = 128
D = 128
NBUF = 4


def _make_propagate(edge_split, pipelined=True):
  e_per_sc = E_PAD // NCORES if edge_split else E_PAD
  k_chunks = e_per_sc // NTILES // CHUNK
  assert k_chunks % 2 == 0

  mesh = plsc.VectorSubcoreMesh(core_axis_name="c", subcore_axis_name="s")

  @functools.partial(
      pl.kernel,
      mesh=mesh,
      out_type=jax.ShapeDtypeStruct((NCORES, N_PAD, D), jnp.float32),
      scratch_types=[
          pltpu.VMEM((CHUNK,), jnp.int32),
          pltpu.VMEM((CHUNK,), jnp.int32),
          pltpu.VMEM((CHUNK,), jnp.int32),
          pltpu.VMEM((CHUNK,), jnp.int32),
          pltpu.VMEM((CHUNK, D), jnp.float32),
          pltpu.VMEM((CHUNK, D), jnp.float32),
          pltpu.SemaphoreType.DMA,
          pltpu.SemaphoreType.DMA,
          pltpu.SemaphoreType.DMA,
          pltpu.SemaphoreType.DMA,
          pltpu.SemaphoreType.DMA,
          pltpu.SemaphoreType.DMA,
          pltpu.VMEM_SHARED((N_PAD, D), jnp.float32),
      ],
  )
  def k(g_hbm, src1d_hbm, dst1d_hbm, out_hbm, sidx0, sidx1, didx0, didx1,
        rows0, rows1, semsi0, semsi1, semdi0, semdi1, semg0, semg1, acc):
    sidxb = (sidx0, sidx1)
    didxb = (didx0, didx1)
    rows = (rows0, rows1)
    semsi = (semsi0, semsi1)
    semdi = (semdi0, semdi1)
    semg = (semg0, semg1)
    c = lax.axis_index("c")
    s = lax.axis_index("s")
    r0 = s * ROWS_PER_TILE
    e_per_tile = e_per_sc // NTILES
    sbase = (c * e_per_sc if edge_split else c * E_PAD) + s * e_per_tile
    dbase = (c * e_per_sc if edge_split else 0) + s * e_per_tile

    def idx_start(j, b):
      pltpu.async_copy(
          src1d_hbm.at[pl.ds(sbase + j * CHUNK, CHUNK)], sidxb[b], semsi[b])
      pltpu.async_copy(
          dst1d_hbm.at[pl.ds(dbase + j * CHUNK, CHUNK)], didxb[b], semdi[b])

    def idx_wait(j, b):
      pltpu.make_async_copy(
          src1d_hbm.at[pl.ds(sbase + j * CHUNK, CHUNK)],
          sidxb[b], semsi[b]).wait()
      pltpu.make_async_copy(
          dst1d_hbm.at[pl.ds(dbase + j * CHUNK, CHUNK)],
          didxb[b], semdi[b]).wait()

    def gather_start(b):
      pltpu.async_copy(g_hbm.at[sidxb[b]], rows[b], semg[b])

    def gather_wait(b):
      pltpu.make_async_copy(g_hbm.at[sidxb[b]], rows[b], semg[b]).wait()

    def scatter(b):
      pltpu.sync_copy(rows[b], acc.at[didxb[b]], add=True)

    if pipelined:
      idx_start(0, 0)
      idx_start(1, 1)
      idx_wait(0, 0)
      gather_start(0)

    tb0 = r0 if edge_split else c * N_PAD + r0
    pltpu.sync_copy(
        g_hbm.at[pl.ds(tb0, ROWS_PER_TILE), :],
        acc.at[pl.ds(r0, ROWS_PER_TILE), :],
    )
    plsc.subcore_barrier()

    if pipelined:
      def step(j, b, last):
        if not last:
          idx_wait(j + 1, 1 - b)
          gather_start(1 - b)
        gather_wait(b)
        scatter(b)
        if not last:
          idx_start(j + 2, b)

      def body(j2, carry):
        j = j2 * 2
        step(j, 0, False)
        step(j + 1, 1, False)
        return carry

      lax.fori_loop(0, k_chunks // 2 - 1, body, 0)

      jt = k_chunks - 2
      idx_wait(jt + 1, 1)
      gather_start(1)
      gather_wait(0)
      scatter(0)
      gather_wait(1)
      scatter(1)
    else:
      def body(j, carry):
        pltpu.sync_copy(
            src1d_hbm.at[pl.ds(sbase + j * CHUNK, CHUNK)], sidxb[0])
        pltpu.sync_copy(
            dst1d_hbm.at[pl.ds(dbase + j * CHUNK, CHUNK)], didxb[0])
        pltpu.async_copy(g_hbm.at[sidxb[0]], rows[0], semg[0]).wait()
        pltpu.sync_copy(rows[0], acc.at[didxb[0]], add=True)
        return carry

      lax.fori_loop(0, k_chunks, body, 0)

    plsc.subcore_barrier()
    pltpu.sync_copy(
        acc.at[pl.ds(r0, ROWS_PER_TILE), :],
        out_hbm.at[c, pl.ds(r0, ROWS_PER_TILE), :],
    )

  return k


def _make_degree():
  e_per_sc = E_PAD // NCORES
  k_chunks = e_per_sc // NTILES // CHUNK

  mesh = plsc.VectorSubcoreMesh(core_axis_name="c", subcore_axis_name="s")

  @functools.partial(
      pl.kernel,
      mesh=mesh,
      out_type=jax.ShapeDtypeStruct((NCORES, N_PAD, D), jnp.float32),
      scratch_types=[
          pltpu.VMEM((k_chunks, CHUNK), jnp.int32),
          pltpu.VMEM((CHUNK, D), jnp.float32),
          pltpu.VMEM_SHARED((N_PAD, D), jnp.float32),
          pltpu.SemaphoreType.DMA,
      ],
  )
  def k(dst2_hbm, ones_hbm, zeros_hbm, out_hbm, didx2d, ones_v, acc, sem):
    c = lax.axis_index("c")
    s = lax.axis_index("s")
    r0 = s * ROWS_PER_TILE
    drow0 = c * (e_per_sc // CHUNK) + s * k_chunks
    pltpu.sync_copy(dst2_hbm.at[pl.ds(drow0, k_chunks), :], didx2d)
    pltpu.sync_copy(zeros_hbm, acc.at[pl.ds(r0, ROWS_PER_TILE), :])
    pltpu.sync_copy(ones_hbm, ones_v)
    plsc.subcore_barrier()

    def body(j, carry):
      pltpu.sync_copy(ones_v, acc.at[didx2d.at[j]], add=True)
      return carry

    lax.fori_loop(0, k_chunks, body, 0)

    plsc.subcore_barrier()
    pltpu.sync_copy(
        acc.at[pl.ds(r0, ROWS_PER_TILE), :],
        out_hbm.at[c, pl.ds(r0, ROWS_PER_TILE), :],
    )

  return k


_prop_esplit = _make_propagate(True, pipelined=False)
_prop_csplit = _make_propagate(False, pipelined=False)
_degree = _make_degree()


_BR = 1024
_GRID = (N_PAD // _BR,)


def _prep_body(d0_ref, d1_ref, x_ref, dinv_ref, g_ref):
  deg = d0_ref[:, 0:1] + d1_ref[:, 0:1] + 1.0
  dinv = lax.rsqrt(deg)
  dinv_ref[...] = dinv
  g_ref[...] = x_ref[...] * dinv


def _prep(d0, d1, x):
  return pl.pallas_call(
      _prep_body,
      grid=_GRID,
      in_specs=[
          pl.BlockSpec((_BR, 128), lambda i: (i, 0)),
          pl.BlockSpec((_BR, 128), lambda i: (i, 0)),
          pl.BlockSpec((_BR, 128), lambda i: (i, 0)),
      ],
      out_specs=[
          pl.BlockSpec((_BR, 1), lambda i: (i, 0)),
          pl.BlockSpec((_BR, 128), lambda i: (i, 0)),
      ],
      out_shape=[
          jax.ShapeDtypeStruct((N_PAD, 1), jnp.float32),
          jax.ShapeDtypeStruct((N_PAD, 128), jnp.float32),
      ],
  )(d0, d1, x)


def _layer1_body(s_ref, g0_ref, dinv_ref, w_ref, b_ref, g_ref):
  dinv = dinv_ref[...]
  u = (s_ref[0, :, :] + s_ref[1, :, :] - g0_ref[...]) * dinv
  y = u @ w_ref[...] + b_ref[...]
  y = jnp.maximum(y, 0.0) * dinv
  g_ref[0, :, :] = y[:, :128]
  g_ref[1, :, :] = y[:, 128:]


def _layer1(s, g0, dinv, w, b):
  return pl.pallas_call(
      _layer1_body,
      grid=_GRID,
      in_specs=[
          pl.BlockSpec((NCORES, _BR, 128), lambda i: (0, i, 0)),
          pl.BlockSpec((_BR, 128), lambda i: (i, 0)),
          pl.BlockSpec((_BR, 1), lambda i: (i, 0)),
          pl.BlockSpec((128, 256), lambda i: (0, 0)),
          pl.BlockSpec((1, 256), lambda i: (0, 0)),
      ],
      out_specs=pl.BlockSpec((NCORES, _BR, 128), lambda i: (0, i, 0)),
      out_shape=jax.ShapeDtypeStruct((NCORES, N_PAD, 128), jnp.float32),
  )(s, g0, dinv, w, b.reshape(1, 256))


def _layer2_body(s_ref, dinv_ref, w_ref, b_ref, g_ref):
  dinv = dinv_ref[...]
  ua = s_ref[0, :, :] * dinv
  ub = s_ref[1, :, :] * dinv
  y = ua @ w_ref[:128, :] + ub @ w_ref[128:, :] + b_ref[...]
  y = jnp.maximum(y, 0.0) * dinv
  g_ref[0, :, :] = y[:, :128]
  g_ref[1, :, :] = y[:, 128:]


def _layer2(s, dinv, w, b):
  return pl.pallas_call(
      _layer2_body,
      grid=_GRID,
      in_specs=[
          pl.BlockSpec((NCORES, _BR, 128), lambda i: (0, i, 0)),
          pl.BlockSpec((_BR, 1), lambda i: (i, 0)),
          pl.BlockSpec((256, 256), lambda i: (0, 0)),
          pl.BlockSpec((1, 256), lambda i: (0, 0)),
      ],
      out_specs=pl.BlockSpec((NCORES, _BR, 128), lambda i: (0, i, 0)),
      out_shape=jax.ShapeDtypeStruct((NCORES, N_PAD, 128), jnp.float32),
  )(s, dinv, w, b.reshape(1, 256))


def _tail_body(s_ref, dinv_ref, w3_ref, b3_ref, wo1_ref, bo1_ref,
               wo2_ref, bo2_ref, out_ref):
  dinv = dinv_ref[...]
  ua = s_ref[0, :, :] * dinv
  ub = s_ref[1, :, :] * dinv
  y = ua @ w3_ref[:128, :] + ub @ w3_ref[128:, :] + b3_ref[...]
  y = jnp.maximum(y, 0.0)
  t = y @ wo1_ref[...] + bo1_ref[...]
  out_ref[...] = t @ wo2_ref[...] + bo2_ref[...]


def _tail(s, dinv, w3, b3, wo1, bo1, wo2, bo2):
  return pl.pallas_call(
      _tail_body,
      grid=_GRID,
      in_specs=[
          pl.BlockSpec((NCORES, _BR, 128), lambda i: (0, i, 0)),
          pl.BlockSpec((_BR, 1), lambda i: (i, 0)),
          pl.BlockSpec((256, 256), lambda i: (0, 0)),
          pl.BlockSpec((1, 256), lambda i: (0, 0)),
          pl.BlockSpec((256, 256), lambda i: (0, 0)),
          pl.BlockSpec((1, 256), lambda i: (0, 0)),
          pl.BlockSpec((256, 128), lambda i: (0, 0)),
          pl.BlockSpec((1, 128), lambda i: (0, 0)),
      ],
      out_specs=pl.BlockSpec((_BR, 128), lambda i: (i, 0)),
      out_shape=jax.ShapeDtypeStruct((N_PAD, 128), jnp.float32),
  )(s, dinv, w3, b3.reshape(1, 256), wo1, bo1.reshape(1, 256),
    wo2, bo2.reshape(1, 128))


def kernel(x, edge_index, W1, b1, W2, b2, W3, b3, Wo1, bo1, Wo2, bo2):
  src = edge_index[0].astype(jnp.int32)
  dst = edge_index[1].astype(jnp.int32)
  pad = E_PAD - N_EDGES
  src_2d = jnp.concatenate(
      [src, jnp.zeros((pad,), jnp.int32)]).reshape(E_PAD // CHUNK, CHUNK)
  dump = N_NODES + jnp.arange(pad, dtype=jnp.int32) % (N_PAD - N_NODES)
  dst_2d = jnp.concatenate([dst, dump]).reshape(E_PAD // CHUNK, CHUNK)
  src_1d = src_2d.reshape(E_PAD)
  dst_1d = dst_2d.reshape(E_PAD)
  src2_1d = jnp.concatenate([src_1d, src_1d + N_PAD])
  x_pad = jnp.pad(x, ((0, N_PAD - N_NODES), (0, 0)))

  ones_in = jnp.ones((CHUNK, D), jnp.float32)
  zeros_in = jnp.zeros((ROWS_PER_TILE, D), jnp.float32)
  d = _degree(dst_2d, ones_in, zeros_in)
  dinv, g0 = _prep(d[0], d[1], x_pad)
  s0 = _prop_esplit(g0, src_1d, dst_1d)
  g1 = _layer1(s0, g0, dinv, W1, b1)
  s1 = _prop_csplit(g1.reshape(2 * N_PAD, 128), src2_1d, dst_1d)
  g2 = _layer2(s1, dinv, W2, b2)
  s2 = _prop_csplit(g2.reshape(2 * N_PAD, 128), src2_1d, dst_1d)
  out = _tail(s2, dinv, W3, b3, Wo1, bo1, Wo2, bo2)
  return out[:N_NODES]


# speedup vs baseline: 1.8290x; 1.8290x over previous
import functools

import jax
import jax.numpy as jnp
from jax import lax
from jax.experimental import pallas as pl
from jax.experimental.pallas import tpu as pltpu
from jax.experimental.pallas import tpu_sc as plsc

N_NODES = 10000
N_PAD = 10240
N_EDGES = 320000
E_PAD = 327680
NTILES = 16
NCORES = 2
ROWS_PER_TILE = N_PAD // NTILES
CHUNK = 128
D = 128
NBUF = 4


def _make_propagate(edge_split, pipelined=True):
  e_per_sc = E_PAD // NCORES if edge_split else E_PAD
  k_chunks = e_per_sc // NTILES // CHUNK
  assert k_chunks % 2 == 0

  mesh = plsc.VectorSubcoreMesh(core_axis_name="c", subcore_axis_name="s")

  @functools.partial(
      pl.kernel,
      mesh=mesh,
      out_type=jax.ShapeDtypeStruct((NCORES, N_PAD, D), jnp.float32),
      scratch_types=[
          pltpu.VMEM((CHUNK,), jnp.int32),
          pltpu.VMEM((CHUNK,), jnp.int32),
          pltpu.VMEM((CHUNK,), jnp.int32),
          pltpu.VMEM((CHUNK,), jnp.int32),
          pltpu.VMEM((CHUNK, D), jnp.float32),
          pltpu.VMEM((CHUNK, D), jnp.float32),
          pltpu.SemaphoreType.DMA,
          pltpu.SemaphoreType.DMA,
          pltpu.SemaphoreType.DMA,
          pltpu.SemaphoreType.DMA,
          pltpu.SemaphoreType.DMA,
          pltpu.SemaphoreType.DMA,
          pltpu.VMEM_SHARED((N_PAD, D), jnp.float32),
      ],
  )
  def k(g_hbm, src1d_hbm, dst1d_hbm, out_hbm, sidx0, sidx1, didx0, didx1,
        rows0, rows1, semsi0, semsi1, semdi0, semdi1, semg0, semg1, acc):
    sidxb = (sidx0, sidx1)
    didxb = (didx0, didx1)
    rows = (rows0, rows1)
    semsi = (semsi0, semsi1)
    semdi = (semdi0, semdi1)
    semg = (semg0, semg1)
    c = lax.axis_index("c")
    s = lax.axis_index("s")
    r0 = s * ROWS_PER_TILE
    e_per_tile = e_per_sc // NTILES
    sbase = (c * e_per_sc if edge_split else c * E_PAD) + s * e_per_tile
    dbase = (c * e_per_sc if edge_split else 0) + s * e_per_tile

    def idx_start(j, b):
      pltpu.async_copy(
          src1d_hbm.at[pl.ds(sbase + j * CHUNK, CHUNK)], sidxb[b], semsi[b])
      pltpu.async_copy(
          dst1d_hbm.at[pl.ds(dbase + j * CHUNK, CHUNK)], didxb[b], semdi[b])

    def idx_wait(j, b):
      pltpu.make_async_copy(
          src1d_hbm.at[pl.ds(sbase + j * CHUNK, CHUNK)],
          sidxb[b], semsi[b]).wait()
      pltpu.make_async_copy(
          dst1d_hbm.at[pl.ds(dbase + j * CHUNK, CHUNK)],
          didxb[b], semdi[b]).wait()

    def gather_start(b):
      pltpu.async_copy(g_hbm.at[sidxb[b]], rows[b], semg[b])

    def gather_wait(b):
      pltpu.make_async_copy(g_hbm.at[sidxb[b]], rows[b], semg[b]).wait()

    def scatter(b):
      pltpu.sync_copy(rows[b], acc.at[didxb[b]], add=True)

    if pipelined:
      idx_start(0, 0)
      idx_start(1, 1)
      idx_wait(0, 0)
      gather_start(0)

    tb0 = r0 if edge_split else c * N_PAD + r0
    pltpu.sync_copy(
        g_hbm.at[pl.ds(tb0, ROWS_PER_TILE), :],
        acc.at[pl.ds(r0, ROWS_PER_TILE), :],
    )
    plsc.subcore_barrier()

    if pipelined:
      def step(j, b, last):
        if not last:
          idx_wait(j + 1, 1 - b)
          gather_start(1 - b)
        gather_wait(b)
        scatter(b)
        if not last:
          idx_start(j + 2, b)

      def body(j2, carry):
        j = j2 * 2
        step(j, 0, False)
        step(j + 1, 1, False)
        return carry

      lax.fori_loop(0, k_chunks // 2 - 1, body, 0)

      jt = k_chunks - 2
      idx_wait(jt + 1, 1)
      gather_start(1)
      gather_wait(0)
      scatter(0)
      gather_wait(1)
      scatter(1)
    else:
      def body(j, carry):
        pltpu.sync_copy(
            src1d_hbm.at[pl.ds(sbase + j * CHUNK, CHUNK)], sidxb[0])
        pltpu.sync_copy(
            dst1d_hbm.at[pl.ds(dbase + j * CHUNK, CHUNK)], didxb[0])
        pltpu.async_copy(g_hbm.at[sidxb[0]], rows[0], semg[0]).wait()
        pltpu.sync_copy(rows[0], acc.at[didxb[0]], add=True)
        return carry

      lax.fori_loop(0, k_chunks, body, 0)

    plsc.subcore_barrier()
    pltpu.sync_copy(
        acc.at[pl.ds(r0, ROWS_PER_TILE), :],
        out_hbm.at[c, pl.ds(r0, ROWS_PER_TILE), :],
    )

  return k


def _make_degree():
  e_per_sc = E_PAD // NCORES
  k_chunks = e_per_sc // NTILES // CHUNK

  mesh = plsc.VectorSubcoreMesh(core_axis_name="c", subcore_axis_name="s")

  @functools.partial(
      pl.kernel,
      mesh=mesh,
      out_type=jax.ShapeDtypeStruct((NCORES, N_PAD, D), jnp.float32),
      scratch_types=[
          pltpu.VMEM((k_chunks, CHUNK), jnp.int32),
          pltpu.VMEM((CHUNK, D), jnp.float32),
          pltpu.VMEM_SHARED((N_PAD, D), jnp.float32),
          pltpu.SemaphoreType.DMA,
      ],
  )
  def k(dst2_hbm, ones_hbm, zeros_hbm, out_hbm, didx2d, ones_v, acc, sem):
    c = lax.axis_index("c")
    s = lax.axis_index("s")
    r0 = s * ROWS_PER_TILE
    drow0 = c * (e_per_sc // CHUNK) + s * k_chunks
    pltpu.sync_copy(dst2_hbm.at[pl.ds(drow0, k_chunks), :], didx2d)
    pltpu.sync_copy(zeros_hbm, acc.at[pl.ds(r0, ROWS_PER_TILE), :])
    pltpu.sync_copy(ones_hbm, ones_v)
    plsc.subcore_barrier()

    def body(j, carry):
      pltpu.sync_copy(ones_v, acc.at[didx2d.at[j]], add=True)
      return carry

    lax.fori_loop(0, k_chunks, body, 0)

    plsc.subcore_barrier()
    pltpu.sync_copy(
        acc.at[pl.ds(r0, ROWS_PER_TILE), :],
        out_hbm.at[c, pl.ds(r0, ROWS_PER_TILE), :],
    )

  return k


_prop_esplit = _make_propagate(True, pipelined=False)
_prop_csplit = _make_propagate(False, pipelined=False)
_degree = _make_degree()


_BR = 1024
_GRID = (N_PAD // _BR,)


def _prep_body(d0_ref, d1_ref, x_ref, dinv_ref, g_ref):
  deg = d0_ref[:, 0:1] + d1_ref[:, 0:1] + 1.0
  dinv = lax.rsqrt(deg)
  dinv_ref[...] = dinv
  g_ref[...] = x_ref[...] * dinv


def _prep(d0, d1, x):
  return pl.pallas_call(
      _prep_body,
      grid=_GRID,
      in_specs=[
          pl.BlockSpec((_BR, 128), lambda i: (i, 0)),
          pl.BlockSpec((_BR, 128), lambda i: (i, 0)),
          pl.BlockSpec((_BR, 128), lambda i: (i, 0)),
      ],
      out_specs=[
          pl.BlockSpec((_BR, 1), lambda i: (i, 0)),
          pl.BlockSpec((_BR, 128), lambda i: (i, 0)),
      ],
      out_shape=[
          jax.ShapeDtypeStruct((N_PAD, 1), jnp.float32),
          jax.ShapeDtypeStruct((N_PAD, 128), jnp.float32),
      ],
  )(d0, d1, x)


def _layer1_body(s_ref, g0_ref, dinv_ref, w_ref, b_ref, g_ref):
  dinv = dinv_ref[...]
  u = (s_ref[0, :, :] + s_ref[1, :, :] - g0_ref[...]) * dinv
  y = u @ w_ref[...] + b_ref[...]
  y = jnp.maximum(y, 0.0) * dinv
  g_ref[0, :, :] = y[:, :128]
  g_ref[1, :, :] = y[:, 128:]


def _layer1(s, g0, dinv, w, b):
  return pl.pallas_call(
      _layer1_body,
      grid=_GRID,
      in_specs=[
          pl.BlockSpec((NCORES, _BR, 128), lambda i: (0, i, 0)),
          pl.BlockSpec((_BR, 128), lambda i: (i, 0)),
          pl.BlockSpec((_BR, 1), lambda i: (i, 0)),
          pl.BlockSpec((128, 256), lambda i: (0, 0)),
          pl.BlockSpec((1, 256), lambda i: (0, 0)),
      ],
      out_specs=pl.BlockSpec((NCORES, _BR, 128), lambda i: (0, i, 0)),
      out_shape=jax.ShapeDtypeStruct((NCORES, N_PAD, 128), jnp.float32),
  )(s, g0, dinv, w, b.reshape(1, 256))


def _layer2_body(s_ref, dinv_ref, w_ref, b_ref, g_ref):
  dinv = dinv_ref[...]
  ua = s_ref[0, :, :] * dinv
  ub = s_ref[1, :, :] * dinv
  y = ua @ w_ref[:128, :] + ub @ w_ref[128:, :] + b_ref[...]
  y = jnp.maximum(y, 0.0) * dinv
  g_ref[0, :, :] = y[:, :128]
  g_ref[1, :, :] = y[:, 128:]


def _layer2(s, dinv, w, b):
  return pl.pallas_call(
      _layer2_body,
      grid=_GRID,
      in_specs=[
          pl.BlockSpec((NCORES, _BR, 128), lambda i: (0, i, 0)),
          pl.BlockSpec((_BR, 1), lambda i: (i, 0)),
          pl.BlockSpec((256, 256), lambda i: (0, 0)),
          pl.BlockSpec((1, 256), lambda i: (0, 0)),
      ],
      out_specs=pl.BlockSpec((NCORES, _BR, 128), lambda i: (0, i, 0)),
      out_shape=jax.ShapeDtypeStruct((NCORES, N_PAD, 128), jnp.float32),
  )(s, dinv, w, b.reshape(1, 256))


def _tail_body(s_ref, dinv_ref, w3_ref, b3_ref, wo1_ref, bo1_ref,
               wo2_ref, bo2_ref, out_ref):
  dinv = dinv_ref[...]
  ua = s_ref[0, :, :] * dinv
  ub = s_ref[1, :, :] * dinv
  y = ua @ w3_ref[:128, :] + ub @ w3_ref[128:, :] + b3_ref[...]
  y = jnp.maximum(y, 0.0)
  t = y @ wo1_ref[...] + bo1_ref[...]
  out_ref[...] = t @ wo2_ref[...] + bo2_ref[...]


def _tail(s, dinv, w3, b3, wo1, bo1, wo2, bo2):
  return pl.pallas_call(
      _tail_body,
      grid=_GRID,
      in_specs=[
          pl.BlockSpec((NCORES, _BR, 128), lambda i: (0, i, 0)),
          pl.BlockSpec((_BR, 1), lambda i: (i, 0)),
          pl.BlockSpec((256, 256), lambda i: (0, 0)),
          pl.BlockSpec((1, 256), lambda i: (0, 0)),
          pl.BlockSpec((256, 256), lambda i: (0, 0)),
          pl.BlockSpec((1, 256), lambda i: (0, 0)),
          pl.BlockSpec((256, 128), lambda i: (0, 0)),
          pl.BlockSpec((1, 128), lambda i: (0, 0)),
      ],
      out_specs=pl.BlockSpec((_BR, 128), lambda i: (i, 0)),
      out_shape=jax.ShapeDtypeStruct((N_PAD, 128), jnp.float32),
  )(s, dinv, w3, b3.reshape(1, 256), wo1, bo1.reshape(1, 256),
    wo2, bo2.reshape(1, 128))


def kernel(x, edge_index, W1, b1, W2, b2, W3, b3, Wo1, bo1, Wo2, bo2):
  src = edge_index[0].astype(jnp.int32)
  dst = edge_index[1].astype(jnp.int32)
  pad = E_PAD - N_EDGES
  dump = N_NODES + jnp.arange(pad, dtype=jnp.int32) % (N_PAD - N_NODES)
  src_2d = jnp.concatenate([src, dump]).reshape(E_PAD // CHUNK, CHUNK)
  dst_2d = jnp.concatenate([dst, dump]).reshape(E_PAD // CHUNK, CHUNK)
  src_1d = src_2d.reshape(E_PAD)
  dst_1d = dst_2d.reshape(E_PAD)
  src2_1d = jnp.concatenate([src_1d, src_1d + N_PAD])
  x_pad = jnp.pad(x, ((0, N_PAD - N_NODES), (0, 0)))

  ones_in = jnp.ones((CHUNK, D), jnp.float32)
  zeros_in = jnp.zeros((ROWS_PER_TILE, D), jnp.float32)
  d = _degree(dst_2d, ones_in, zeros_in)
  dinv, g0 = _prep(d[0], d[1], x_pad)
  s0 = _prop_esplit(g0, src_1d, dst_1d)
  g1 = _layer1(s0, g0, dinv, W1, b1)
  s1 = _prop_csplit(g1.reshape(2 * N_PAD, 128), src2_1d, dst_1d)
  g2 = _layer2(s1, dinv, W2, b2)
  s2 = _prop_csplit(g2.reshape(2 * N_PAD, 128), src2_1d, dst_1d)
  out = _tail(s2, dinv, W3, b3, Wo1, bo1, Wo2, bo2)
  return out[:N_NODES]
